# baseline (speedup 1.0000x reference)
import functools

import jax
import jax.numpy as jnp
from jax import lax
from jax.experimental import pallas as pl
from jax.experimental.pallas import tpu as pltpu
from jax.experimental.pallas import tpu_sc as plsc

BLK = 128


@functools.lru_cache(maxsize=None)
def _make_gather(batch: int, seq: int, embed_d: int):
    info = plsc.get_sparse_core_info()
    num_cores, num_subcores = info.num_cores, info.num_subcores
    n_workers = num_cores * num_subcores
    n_blk = batch // BLK
    n_grp = n_workers // n_blk
    s_per_w = seq // n_grp
    mesh = plsc.VectorSubcoreMesh(core_axis_name="c", subcore_axis_name="s")

    @functools.partial(
        pl.kernel,
        mesh=mesh,
        out_type=jax.ShapeDtypeStruct((seq, batch, embed_d), jnp.float32),
        scratch_types=[
            pltpu.VMEM((seq, BLK), jnp.int32),
            pltpu.VMEM((s_per_w, BLK, embed_d), jnp.float32),
            pltpu.SemaphoreType.DMA,
            pltpu.SemaphoreType.DMA,
        ],
    )
    def gather_kernel(table_hbm, idx_hbm, out_hbm, idx_v, rows_v, sem_g, sem_s):
        wid = lax.axis_index("s") * num_cores + lax.axis_index("c")
        q = wid % n_blk
        r = wid // n_blk
        pltpu.sync_copy(idx_hbm.at[:, pl.ds(q * BLK, BLK)], idx_v)
        gathers = [
            pltpu.async_copy(
                table_hbm.at[idx_v.at[r * s_per_w + j]], rows_v.at[j], sem_g
            )
            for j in range(s_per_w)
        ]
        stores = []
        for j in range(s_per_w):
            gathers[j].wait()
            stores.append(
                pltpu.async_copy(
                    rows_v.at[j],
                    out_hbm.at[r * s_per_w + j, pl.ds(q * BLK, BLK)],
                    sem_s,
                )
            )
        for s in stores:
            s.wait()

    return gather_kernel


@functools.lru_cache(maxsize=None)
def _make_tc_lookup(batch: int, seq: int, vocab: int, embed_d: int):
    def body(idx_ref, w_ref, out_ref):
        s = pl.program_id(0)
        idx_col = idx_ref[pl.ds(s, 1), :].reshape(batch, 1)
        viota = lax.broadcasted_iota(jnp.int32, (batch, vocab), 1)
        oh = (viota == idx_col).astype(jnp.bfloat16)
        res = jnp.dot(
            oh, w_ref[...], preferred_element_type=jnp.float32
        )
        out_ref[...] = res.reshape(1, batch, embed_d)

    return pl.pallas_call(
        body,
        grid=(seq,),
        in_specs=[
            pl.BlockSpec((seq, batch), lambda s: (0, 0)),
            pl.BlockSpec((vocab, embed_d), lambda s: (0, 0)),
        ],
        out_specs=pl.BlockSpec((1, batch, embed_d), lambda s: (s, 0, 0)),
        out_shape=jax.ShapeDtypeStruct((seq, batch, embed_d), jnp.float32),
        compiler_params=pltpu.CompilerParams(
            dimension_semantics=("arbitrary",)
        ),
    )


def kernel(arr, W_s):
    batch, seq = arr.shape
    vocab, embed_d = W_s.shape
    out = _make_tc_lookup(batch, seq, vocab, embed_d)(
        arr.T.astype(jnp.int32), W_s.astype(jnp.bfloat16)
    )
    return out.transpose(1, 0, 2)


# speedup vs baseline: 1.0940x; 1.0940x over previous
import functools

import jax
import jax.numpy as jnp
from jax import lax
from jax.experimental import pallas as pl
from jax.experimental.pallas import tpu as pltpu
from jax.experimental.pallas import tpu_sc as plsc

BLK = 128


@functools.lru_cache(maxsize=None)
def _make_gather(batch: int, seq: int, embed_d: int):
    info = plsc.get_sparse_core_info()
    num_cores, num_subcores = info.num_cores, info.num_subcores
    n_workers = num_cores * num_subcores
    n_blk = batch // BLK
    n_grp = n_workers // n_blk
    s_per_w = seq // n_grp
    mesh = plsc.VectorSubcoreMesh(core_axis_name="c", subcore_axis_name="s")

    @functools.partial(
        pl.kernel,
        mesh=mesh,
        out_type=jax.ShapeDtypeStruct((seq, batch, embed_d), jnp.float32),
        scratch_types=[
            pltpu.VMEM((seq, BLK), jnp.int32),
            pltpu.VMEM((s_per_w, BLK, embed_d), jnp.float32),
            pltpu.SemaphoreType.DMA,
            pltpu.SemaphoreType.DMA,
        ],
    )
    def gather_kernel(table_hbm, idx_hbm, out_hbm, idx_v, rows_v, sem_g, sem_s):
        wid = lax.axis_index("s") * num_cores + lax.axis_index("c")
        q = wid % n_blk
        r = wid // n_blk
        pltpu.sync_copy(idx_hbm.at[:, pl.ds(q * BLK, BLK)], idx_v)
        gathers = [
            pltpu.async_copy(
                table_hbm.at[idx_v.at[r * s_per_w + j]], rows_v.at[j], sem_g
            )
            for j in range(s_per_w)
        ]
        stores = []
        for j in range(s_per_w):
            gathers[j].wait()
            stores.append(
                pltpu.async_copy(
                    rows_v.at[j],
                    out_hbm.at[r * s_per_w + j, pl.ds(q * BLK, BLK)],
                    sem_s,
                )
            )
        for s in stores:
            s.wait()

    return gather_kernel


@functools.lru_cache(maxsize=None)
def _make_tc_lookup(batch: int, seq: int, vocab: int, embed_d: int):
    def body(idx_ref, w_ref, out_ref, wbf_ref):
        s = pl.program_id(0)

        @pl.when(s == 0)
        def _():
            wbf_ref[...] = w_ref[...].astype(jnp.bfloat16)

        idx_col = idx_ref[pl.ds(s, 1), :].reshape(batch, 1)
        viota = lax.broadcasted_iota(jnp.int32, (batch, vocab), 1)
        oh = (viota == idx_col).astype(jnp.bfloat16)
        res = jnp.dot(
            oh, wbf_ref[...], preferred_element_type=jnp.float32
        )
        out_ref[...] = res.reshape(1, batch, embed_d)

    return pl.pallas_call(
        body,
        grid=(seq,),
        in_specs=[
            pl.BlockSpec((seq, batch), lambda s: (0, 0)),
            pl.BlockSpec((vocab, embed_d), lambda s: (0, 0)),
        ],
        out_specs=pl.BlockSpec((1, batch, embed_d), lambda s: (s, 0, 0)),
        out_shape=jax.ShapeDtypeStruct((seq, batch, embed_d), jnp.float32),
        scratch_shapes=[pltpu.VMEM((vocab, embed_d), jnp.bfloat16)],
        compiler_params=pltpu.CompilerParams(
            dimension_semantics=("arbitrary",)
        ),
    )


def kernel(arr, W_s):
    batch, seq = arr.shape
    vocab, embed_d = W_s.shape
    out = _make_tc_lookup(batch, seq, vocab, embed_d)(
        arr.T.astype(jnp.int32), W_s
    )
    return out.transpose(1, 0, 2)


# speedup vs baseline: 1.0942x; 1.0002x over previous
import functools

import jax
import jax.numpy as jnp
from jax import lax
from jax.experimental import pallas as pl
from jax.experimental.pallas import tpu as pltpu


@functools.lru_cache(maxsize=None)
def _make_tc_lookup(batch: int, seq: int, vocab: int, embed_d: int):
    def body(idx_ref, w_ref, out_ref, wbf_ref):
        s = pl.program_id(0)

        @pl.when(s == 0)
        def _():
            wbf_ref[...] = w_ref[...].astype(jnp.bfloat16)

        idx_col = idx_ref[pl.ds(s, 1), :].reshape(batch, 1)
        viota = lax.broadcasted_iota(jnp.int32, (batch, vocab), 1)
        oh = (viota == idx_col).astype(jnp.bfloat16)
        res = jnp.dot(
            oh, wbf_ref[...], preferred_element_type=jnp.float32
        )
        out_ref[...] = res.reshape(1, batch, embed_d)

    return pl.pallas_call(
        body,
        grid=(seq,),
        in_specs=[
            pl.BlockSpec((seq, batch), lambda s: (0, 0)),
            pl.BlockSpec((vocab, embed_d), lambda s: (0, 0)),
        ],
        out_specs=pl.BlockSpec((1, batch, embed_d), lambda s: (s, 0, 0)),
        out_shape=jax.ShapeDtypeStruct((seq, batch, embed_d), jnp.float32),
        scratch_shapes=[pltpu.VMEM((vocab, embed_d), jnp.bfloat16)],
        compiler_params=pltpu.CompilerParams(
            dimension_semantics=("arbitrary",)
        ),
    )


def kernel(arr, W_s):
    batch, seq = arr.shape
    vocab, embed_d = W_s.shape
    out = _make_tc_lookup(batch, seq, vocab, embed_d)(
        arr.T.astype(jnp.int32), W_s
    )
    return out.transpose(1, 0, 2)
